# Initial kernel scaffold; baseline (speedup 1.0000x reference)
import jax
import jax.numpy as jnp
from jax.experimental import pallas as pl


def kernel(input):
    raise NotImplementedError("write your pallas kernel here")


# speedup vs baseline: 156.3331x; 156.3331x over previous
import functools

import jax
import jax.numpy as jnp
import numpy as np
from jax.experimental import pallas as pl
from jax.experimental.pallas import tpu as pltpu

_BITS = 4
_K = 2 ** _BITS
_ITERS = 10

_ROWS = 147
_SUB = 256
_LANE = 128
_BR = 3
_NC = _ROWS // _BR


def _lloyd_kernel(c0_ref, x_ref, cout_ref, cen_ref, accC_ref, accS_ref):
    it = pl.program_id(0)
    chunk = pl.program_id(1)

    @pl.when(jnp.logical_and(it == 0, chunk == 0))
    def _init():
        for k in range(_K):
            cen_ref[k] = c0_ref[k]

    @pl.when(chunk == 0)
    def _zero():
        for j in range(_K):
            accC_ref[j] = 0.0
            accS_ref[j] = 0.0

    x = x_ref[...]
    c = [cen_ref[k] for k in range(_K)]
    for j in range(_K - 1):
        m = (c[j] + c[j + 1]) * 0.5
        mask = x <= m
        accC_ref[j] += jnp.sum(jnp.where(mask, 1.0, 0.0))
        accS_ref[j] += jnp.sum(jnp.where(mask, x, 0.0))
    accS_ref[_K - 1] += jnp.sum(x)

    @pl.when(chunk == _NC - 1)
    def _update():
        n_total = float(_ROWS * _SUB * _LANE)
        prev_c = 0.0
        prev_s = 0.0
        for k in range(_K):
            Ck = accC_ref[k] if k < _K - 1 else n_total
            Sk = accS_ref[k]
            cnt = Ck - prev_c
            s = Sk - prev_s
            new_c = jnp.where(cnt > 0.0, s / jnp.maximum(cnt, 1.0), cen_ref[k])
            cen_ref[k] = new_c
            cout_ref[k] = new_c
            prev_c, prev_s = Ck, Sk


def _assign_kernel(cen_ref, x_ref, out_ref):
    x = x_ref[...]
    c = [cen_ref[k] for k in range(_K)]
    out = jnp.full(x.shape, c[0], dtype=jnp.float32)
    for j in range(_K - 1):
        m = (c[j] + c[j + 1]) * 0.5
        out = jnp.where(x > m, c[j + 1], out)
    out_ref[...] = out


@functools.partial(jax.jit)
def kernel(input):
    shape = input.shape
    flat = input.reshape(-1)
    n = flat.shape[0]
    init_idx = jnp.linspace(0, n - 1, _K).astype(jnp.int32)
    c0 = jnp.sort(flat[init_idx])

    x3 = flat.reshape(_ROWS, _SUB, _LANE)

    centers = pl.pallas_call(
        _lloyd_kernel,
        grid=(_ITERS, _NC),
        in_specs=[
            pl.BlockSpec(memory_space=pltpu.SMEM),
            pl.BlockSpec((_BR, _SUB, _LANE), lambda i, c: (c, 0, 0)),
        ],
        out_specs=pl.BlockSpec(memory_space=pltpu.SMEM),
        out_shape=jax.ShapeDtypeStruct((_K,), jnp.float32),
        scratch_shapes=[
            pltpu.SMEM((_K,), jnp.float32),
            pltpu.SMEM((_K,), jnp.float32),
            pltpu.SMEM((_K,), jnp.float32),
        ],
    )(c0, x3)

    out = pl.pallas_call(
        _assign_kernel,
        grid=(_NC,),
        in_specs=[
            pl.BlockSpec(memory_space=pltpu.SMEM),
            pl.BlockSpec((_BR, _SUB, _LANE), lambda c: (c, 0, 0)),
        ],
        out_specs=pl.BlockSpec((_BR, _SUB, _LANE), lambda c: (c, 0, 0)),
        out_shape=jax.ShapeDtypeStruct((_ROWS, _SUB, _LANE), jnp.float32),
    )(centers, x3)

    return out.reshape(shape)


# speedup vs baseline: 347.3924x; 2.2221x over previous
import functools

import jax
import jax.numpy as jnp
from jax import lax
from jax.experimental import pallas as pl
from jax.experimental.pallas import tpu as pltpu
from jax.experimental.pallas import tpu_sc as plsc

_K = 16
_ITERS = 10
_B = 65536
_BS = 512
_BL = 128

_ROWS = 147
_SUB = 256
_LANE = 128
_BR = 3
_NC = _ROWS // _BR
_N = _ROWS * _SUB * _LANE

_NTILES = 32
_PER_TILE = _N // _NTILES
_CH = 3072
_NCHUNK = _PER_TILE // _CH


def _minmax_kernel(x_ref, p_ref, mm_ref):
    step = pl.program_id(0)
    x = x_ref[...]

    @pl.when(step == 0)
    def _init():
        mm_ref[0] = jnp.inf
        mm_ref[1] = -jnp.inf

    mm_ref[0] = jnp.minimum(mm_ref[0], jnp.min(x))
    mm_ref[1] = jnp.maximum(mm_ref[1], jnp.max(x))

    @pl.when(step == _NC - 1)
    def _fin():
        lo = mm_ref[0]
        hi = mm_ref[1]
        invw = _B / jnp.maximum(hi - lo, 1e-30)
        for i in range(16):
            p_ref[i] = lo
            p_ref[16 + i] = invw


def _sc_hist_kernel(x_hbm, p_hbm, out_hbm, hist, buf0, buf1, pbuf,
                    sem0, sem1, psem):
    wid = lax.axis_index("s") * 2 + lax.axis_index("c")

    pltpu.async_copy(p_hbm, pbuf, psem).wait()
    lo = pbuf[pl.ds(0, 16)]
    invw = pbuf[pl.ds(16, 16)]

    def _zero(i, carry):
        hist[pl.ds(i * 16, 16)] = jnp.zeros((16,), jnp.int32)
        return carry

    lax.fori_loop(0, _B // 16, _zero, 0)

    ones = jnp.full((16,), 1, jnp.int32)
    maxbin = jnp.full((16,), _B - 1, jnp.int32)
    zeroi = jnp.zeros((16,), jnp.int32)

    def _process(buf):
        def body(k, carry):
            v = buf[pl.ds(k * 16, 16)]
            t = (v - lo) * invw
            bi = t.astype(jnp.int32)
            bi = jnp.minimum(jnp.maximum(bi, zeroi), maxbin)
            plsc.addupdate_scatter(hist, [bi], ones)
            return carry

        lax.fori_loop(0, _CH // 16, body, 0)

    base0 = pl.multiple_of(wid * _PER_TILE, 512)
    bufs = (buf0, buf1)
    sems = (sem0, sem1)
    handles = [None, None]
    handles[0] = pltpu.async_copy(x_hbm.at[pl.ds(base0, _CH)], buf0, sem0)
    for g in range(_NCHUNK):
        cur = g % 2
        if g + 1 < _NCHUNK:
            nxt = (g + 1) % 2
            handles[nxt] = pltpu.async_copy(
                x_hbm.at[pl.ds(base0 + (g + 1) * _CH, _CH)],
                bufs[nxt], sems[nxt])
        handles[cur].wait()
        _process(bufs[cur])

    pltpu.sync_copy(hist, out_hbm.at[wid])


_sc_hist = functools.partial(
    pl.kernel,
    out_type=jax.ShapeDtypeStruct((_NTILES, _B), jnp.int32),
    scratch_types=[
        pltpu.VMEM((_B,), jnp.int32),
        pltpu.VMEM((_CH,), jnp.float32),
        pltpu.VMEM((_CH,), jnp.float32),
        pltpu.VMEM((32,), jnp.float32),
        pltpu.SemaphoreType.DMA,
        pltpu.SemaphoreType.DMA,
        pltpu.SemaphoreType.DMA,
    ],
    mesh=plsc.VectorSubcoreMesh(core_axis_name="c", subcore_axis_name="s"),
    compiler_params=pltpu.CompilerParams(needs_layout_passes=False),
)(_sc_hist_kernel)


def _lloyd_assign_kernel(hists_ref, c0_ref, p_ref, x_ref, out_ref,
                         cen_ref, cnt_ref, cntbc_ref, bc_ref):
    step = pl.program_id(0)

    @pl.when(step == 0)
    def _lloyd():
        lo = p_ref[0]
        invw = p_ref[16]
        w = 1.0 / invw
        cnt = hists_ref[0].astype(jnp.float32)
        for t in range(1, _NTILES):
            cnt += hists_ref[t].astype(jnp.float32)
        gidx = (lax.broadcasted_iota(jnp.int32, (_BS, _BL), 0) * _BL
                + lax.broadcasted_iota(jnp.int32, (_BS, _BL), 1)
                ).astype(jnp.float32)
        bc = lo + (gidx + 0.5) * w
        cntbc = cnt * bc
        cnt_ref[...] = cnt
        cntbc_ref[...] = cntbc
        bc_ref[...] = bc
        tot_c = jnp.sum(cnt)
        tot_s = jnp.sum(cntbc)

        c = [c0_ref[k] for k in range(_K)]
        for _ in range(_ITERS):
            Cs = []
            Ss = []
            for j in range(_K - 1):
                m = (c[j] + c[j + 1]) * 0.5
                wg = jnp.clip((m - bc) * invw + 0.5, 0.0, 1.0)
                Cs.append(jnp.sum(cnt * wg))
                Ss.append(jnp.sum(cntbc * wg))
            Cs.append(tot_c)
            Ss.append(tot_s)
            prev_c = 0.0
            prev_s = 0.0
            newc = []
            for k in range(_K):
                ck = Cs[k] - prev_c
                sk = Ss[k] - prev_s
                newc.append(jnp.where(ck > 0.0, sk / jnp.maximum(ck, 1.0), c[k]))
                prev_c, prev_s = Cs[k], Ss[k]
            c = newc
        for k in range(_K):
            cen_ref[k] = c[k]

    @pl.when(step > 0)
    def _assign():
        x = x_ref[...]
        c = [cen_ref[k] for k in range(_K)]
        out = jnp.full(x.shape, c[0], dtype=jnp.float32)
        for j in range(_K - 1):
            m = (c[j] + c[j + 1]) * 0.5
            out = jnp.where(x > m, c[j + 1], out)
        out_ref[...] = out


@functools.partial(jax.jit)
def kernel(input):
    shape = input.shape
    flat = input.reshape(-1)
    n = flat.shape[0]
    init_idx = jnp.linspace(0, n - 1, _K).astype(jnp.int32)
    c0 = jnp.sort(flat[init_idx])

    x3 = flat.reshape(_ROWS, _SUB, _LANE)

    params = pl.pallas_call(
        _minmax_kernel,
        grid=(_NC,),
        in_specs=[pl.BlockSpec((_BR, _SUB, _LANE), lambda c: (c, 0, 0))],
        out_specs=pl.BlockSpec(memory_space=pltpu.SMEM),
        out_shape=jax.ShapeDtypeStruct((32,), jnp.float32),
        scratch_shapes=[pltpu.SMEM((2,), jnp.float32)],
    )(x3)

    partials = _sc_hist(flat, params)

    out = pl.pallas_call(
        _lloyd_assign_kernel,
        grid=(_NC + 1,),
        in_specs=[
            pl.BlockSpec((_NTILES, _BS, _BL), lambda i: (0, 0, 0)),
            pl.BlockSpec(memory_space=pltpu.SMEM),
            pl.BlockSpec(memory_space=pltpu.SMEM),
            pl.BlockSpec((_BR, _SUB, _LANE),
                         lambda i: (jnp.maximum(i - 1, 0), 0, 0)),
        ],
        out_specs=pl.BlockSpec((_BR, _SUB, _LANE),
                               lambda i: (jnp.maximum(i - 1, 0), 0, 0)),
        out_shape=jax.ShapeDtypeStruct((_ROWS, _SUB, _LANE), jnp.float32),
        scratch_shapes=[
            pltpu.SMEM((_K,), jnp.float32),
            pltpu.VMEM((_BS, _BL), jnp.float32),
            pltpu.VMEM((_BS, _BL), jnp.float32),
            pltpu.VMEM((_BS, _BL), jnp.float32),
        ],
    )(partials.reshape(_NTILES, _BS, _BL), c0, params, x3)

    return out.reshape(shape)


# speedup vs baseline: 368.3371x; 1.0603x over previous
import functools

import jax
import jax.numpy as jnp
from jax import lax
from jax.experimental import pallas as pl
from jax.experimental.pallas import tpu as pltpu
from jax.experimental.pallas import tpu_sc as plsc

_K = 16
_ITERS = 10
_B = 65536
_BS = 512
_BL = 128

_ROWS = 147
_SUB = 256
_LANE = 128
_BR = 3
_NC = _ROWS // _BR
_N = _ROWS * _SUB * _LANE

_NTILES = 32
_PER_TILE = _N // _NTILES
_CH = 3072
_NCHUNK = _PER_TILE // _CH


def _minmax_kernel(x_ref, p_ref, mm_ref):
    step = pl.program_id(0)
    x = x_ref[...]

    @pl.when(step == 0)
    def _init():
        mm_ref[0] = jnp.inf
        mm_ref[1] = -jnp.inf

    mm_ref[0] = jnp.minimum(mm_ref[0], jnp.min(x))
    mm_ref[1] = jnp.maximum(mm_ref[1], jnp.max(x))

    @pl.when(step == _NC - 1)
    def _fin():
        lo = mm_ref[0]
        hi = mm_ref[1]
        invw = _B / jnp.maximum(hi - lo, 1e-30)
        for i in range(16):
            p_ref[i] = lo
            p_ref[16 + i] = invw


def _sc_hist_kernel(x_hbm, p_hbm, out_hbm, hist, buf0, buf1, pbuf,
                    sem0, sem1, psem):
    wid = lax.axis_index("s") * 2 + lax.axis_index("c")

    pltpu.async_copy(p_hbm, pbuf, psem).wait()
    lo = pbuf[pl.ds(0, 16)]
    invw = pbuf[pl.ds(16, 16)]

    zvec = jnp.zeros((16,), jnp.int32)

    def _zero(i, carry):
        for u in range(16):
            hist[pl.ds(i * 256 + u * 16, 16)] = zvec
        return carry

    lax.fori_loop(0, _B // 256, _zero, 0)

    ones = jnp.full((16,), 1, jnp.int32)
    maxbin = jnp.full((16,), _B - 1, jnp.int32)

    def _process(buf):
        def body(k, carry):
            for u in range(8):
                v = buf[pl.ds(k * 128 + u * 16, 16)]
                t = (v - lo) * invw
                bi = jnp.minimum(t.astype(jnp.int32), maxbin)
                plsc.addupdate_scatter(hist, [bi], ones)
            return carry

        lax.fori_loop(0, _CH // 128, body, 0)

    base0 = pl.multiple_of(wid * _PER_TILE, 512)
    bufs = (buf0, buf1)
    sems = (sem0, sem1)
    handles = [None, None]
    handles[0] = pltpu.async_copy(x_hbm.at[pl.ds(base0, _CH)], buf0, sem0)
    for g in range(_NCHUNK):
        cur = g % 2
        if g + 1 < _NCHUNK:
            nxt = (g + 1) % 2
            handles[nxt] = pltpu.async_copy(
                x_hbm.at[pl.ds(base0 + (g + 1) * _CH, _CH)],
                bufs[nxt], sems[nxt])
        handles[cur].wait()
        _process(bufs[cur])

    pltpu.sync_copy(hist, out_hbm.at[wid])


_sc_hist = functools.partial(
    pl.kernel,
    out_type=jax.ShapeDtypeStruct((_NTILES, _B), jnp.int32),
    scratch_types=[
        pltpu.VMEM((_B,), jnp.int32),
        pltpu.VMEM((_CH,), jnp.float32),
        pltpu.VMEM((_CH,), jnp.float32),
        pltpu.VMEM((32,), jnp.float32),
        pltpu.SemaphoreType.DMA,
        pltpu.SemaphoreType.DMA,
        pltpu.SemaphoreType.DMA,
    ],
    mesh=plsc.VectorSubcoreMesh(core_axis_name="c", subcore_axis_name="s"),
    compiler_params=pltpu.CompilerParams(needs_layout_passes=False),
)(_sc_hist_kernel)


def _lloyd_assign_kernel(hists_ref, c0_ref, p_ref, x_ref, out_ref,
                         cen_ref, cnt_ref, cntbc_ref, bc_ref):
    step = pl.program_id(0)

    @pl.when(step == 0)
    def _lloyd():
        lo = p_ref[0]
        invw = p_ref[16]
        w = 1.0 / invw
        cnt = hists_ref[0].astype(jnp.float32)
        for t in range(1, _NTILES):
            cnt += hists_ref[t].astype(jnp.float32)
        gidx = (lax.broadcasted_iota(jnp.int32, (_BS, _BL), 0) * _BL
                + lax.broadcasted_iota(jnp.int32, (_BS, _BL), 1)
                ).astype(jnp.float32)
        bc = lo + (gidx + 0.5) * w
        cntbc = cnt * bc
        cnt_ref[...] = cnt
        cntbc_ref[...] = cntbc
        bc_ref[...] = bc
        tot_c = jnp.sum(cnt)
        tot_s = jnp.sum(cntbc)

        c = [c0_ref[k] for k in range(_K)]
        for _ in range(_ITERS):
            Cs = []
            Ss = []
            for j in range(_K - 1):
                m = (c[j] + c[j + 1]) * 0.5
                wg = jnp.clip((m - bc) * invw + 0.5, 0.0, 1.0)
                Cs.append(jnp.sum(cnt * wg))
                Ss.append(jnp.sum(cntbc * wg))
            Cs.append(tot_c)
            Ss.append(tot_s)
            prev_c = 0.0
            prev_s = 0.0
            newc = []
            for k in range(_K):
                ck = Cs[k] - prev_c
                sk = Ss[k] - prev_s
                newc.append(jnp.where(ck > 0.0, sk / jnp.maximum(ck, 1.0), c[k]))
                prev_c, prev_s = Cs[k], Ss[k]
            c = newc
        for k in range(_K):
            cen_ref[k] = c[k]

    @pl.when(step > 0)
    def _assign():
        x = x_ref[...]
        c = [cen_ref[k] for k in range(_K)]
        out = jnp.full(x.shape, c[0], dtype=jnp.float32)
        for j in range(_K - 1):
            m = (c[j] + c[j + 1]) * 0.5
            out = jnp.where(x > m, c[j + 1], out)
        out_ref[...] = out


@functools.partial(jax.jit)
def kernel(input):
    shape = input.shape
    flat = input.reshape(-1)
    n = flat.shape[0]
    init_idx = jnp.linspace(0, n - 1, _K).astype(jnp.int32)
    c0 = jnp.sort(flat[init_idx])

    x3 = flat.reshape(_ROWS, _SUB, _LANE)

    params = pl.pallas_call(
        _minmax_kernel,
        grid=(_NC,),
        in_specs=[pl.BlockSpec((_BR, _SUB, _LANE), lambda c: (c, 0, 0))],
        out_specs=pl.BlockSpec(memory_space=pltpu.SMEM),
        out_shape=jax.ShapeDtypeStruct((32,), jnp.float32),
        scratch_shapes=[pltpu.SMEM((2,), jnp.float32)],
    )(x3)

    partials = _sc_hist(flat, params)

    out = pl.pallas_call(
        _lloyd_assign_kernel,
        grid=(_NC + 1,),
        in_specs=[
            pl.BlockSpec((_NTILES, _BS, _BL), lambda i: (0, 0, 0)),
            pl.BlockSpec(memory_space=pltpu.SMEM),
            pl.BlockSpec(memory_space=pltpu.SMEM),
            pl.BlockSpec((_BR, _SUB, _LANE),
                         lambda i: (jnp.maximum(i - 1, 0), 0, 0)),
        ],
        out_specs=pl.BlockSpec((_BR, _SUB, _LANE),
                               lambda i: (jnp.maximum(i - 1, 0), 0, 0)),
        out_shape=jax.ShapeDtypeStruct((_ROWS, _SUB, _LANE), jnp.float32),
        scratch_shapes=[
            pltpu.SMEM((_K,), jnp.float32),
            pltpu.VMEM((_BS, _BL), jnp.float32),
            pltpu.VMEM((_BS, _BL), jnp.float32),
            pltpu.VMEM((_BS, _BL), jnp.float32),
        ],
    )(partials.reshape(_NTILES, _BS, _BL), c0, params, x3)

    return out.reshape(shape)
